# Initial kernel scaffold; baseline (speedup 1.0000x reference)
import jax
import jax.numpy as jnp
from jax.experimental import pallas as pl


def kernel(x, Wg, bg, W1, b1, W2, b2):
    raise NotImplementedError("write your pallas kernel here")


# speedup vs baseline: 2.0793x; 2.0793x over previous
import functools
import jax
import jax.numpy as jnp
from jax.experimental import pallas as pl
from jax.experimental.pallas import tpu as pltpu

_H = 1024
_E = 8
_K = 2
_F = 4096
_T = 4096

_B = 256
_P = _T * _K + _E * _B
_NB = _P // _B
_FB = 1024
_NF = _F // _FB

_NEG = -3e38


def _gating_body(x_ref, wg_ref, bg_ref, out_ref):
    s = jnp.dot(x_ref[...], wg_ref[...], preferred_element_type=jnp.float32)
    s = s + bg_ref[...]
    col = jax.lax.broadcasted_iota(jnp.int32, s.shape, 1)
    s = jnp.where(col < _E, s, _NEG)
    m1 = jnp.max(s, axis=1, keepdims=True)
    i1 = jnp.min(jnp.where(s == m1, col, 128), axis=1, keepdims=True)
    s2 = jnp.where(col == i1, _NEG, s)
    m2 = jnp.max(s2, axis=1, keepdims=True)
    i2 = jnp.min(jnp.where(s2 == m2, col, 128), axis=1, keepdims=True)
    z = jnp.exp(m2 - m1)
    w1 = 1.0 / (1.0 + z)
    w2 = z / (1.0 + z)
    out = jnp.where(col == 0, i1.astype(jnp.float32),
          jnp.where(col == 1, i2.astype(jnp.float32),
          jnp.where(col == 2, w1,
          jnp.where(col == 3, w2, 0.0))))
    out_ref[...] = out


def _gating(x, Wg, bg):
    tb = 1024
    wgp = jnp.zeros((_H, 128), jnp.float32).at[:, :_E].set(Wg)
    bgp = jnp.zeros((1, 128), jnp.float32).at[0, :_E].set(bg)
    out = pl.pallas_call(
        _gating_body,
        grid=(_T // tb,),
        in_specs=[
            pl.BlockSpec((tb, _H), lambda i: (i, 0)),
            pl.BlockSpec((_H, 128), lambda i: (0, 0)),
            pl.BlockSpec((1, 128), lambda i: (0, 0)),
        ],
        out_specs=pl.BlockSpec((tb, 128), lambda i: (i, 0)),
        out_shape=jax.ShapeDtypeStruct((_T, 128), jnp.float32),
    )(x, wgp, bgp)
    idx = out[:, :2].astype(jnp.int32)
    w = out[:, 2:4]
    return idx, w


def _ffn_body(meta_ref, xs_ref, w1_ref, b1_ref, w2_ref, b2_ref, ys_ref):
    b = pl.program_id(0)
    f = pl.program_id(1)
    nvalid = meta_ref[_NB]

    @pl.when(b < nvalid)
    def _():
        h = jnp.dot(xs_ref[...], w1_ref[0], preferred_element_type=jnp.float32)
        h = jnp.maximum(h + b1_ref[0], 0.0)
        acc = jnp.dot(h, w2_ref[0], preferred_element_type=jnp.float32)

        @pl.when(f == 0)
        def _():
            ys_ref[...] = acc + b2_ref[0]

        @pl.when(f > 0)
        def _():
            ys_ref[...] += acc


def _grouped_ffn(meta, xs, W1, b1, W2, b2):
    grid_spec = pltpu.PrefetchScalarGridSpec(
        num_scalar_prefetch=1,
        grid=(_NB, _NF),
        in_specs=[
            pl.BlockSpec((_B, _H), lambda b, f, m: (b, 0)),
            pl.BlockSpec((1, _H, _FB), lambda b, f, m: (m[b], 0, f)),
            pl.BlockSpec((1, 1, _FB), lambda b, f, m: (m[b], 0, f)),
            pl.BlockSpec((1, _FB, _H), lambda b, f, m: (m[b], f, 0)),
            pl.BlockSpec((1, 1, _H), lambda b, f, m: (m[b], 0, 0)),
        ],
        out_specs=pl.BlockSpec((_B, _H), lambda b, f, m: (b, 0)),
    )
    return pl.pallas_call(
        _ffn_body,
        grid_spec=grid_spec,
        out_shape=jax.ShapeDtypeStruct((_P, _H), jnp.float32),
    )(meta, xs, W1, b1.reshape(_E, 1, _F), W2, b2.reshape(_E, 1, _H))


@jax.jit
def kernel(x, Wg, bg, W1, b1, W2, b2):
    idx, w = _gating(x, Wg, bg)

    flat_e = idx.reshape(-1)
    counts = jnp.sum(flat_e[:, None] == jnp.arange(_E)[None, :], axis=0)
    padded = ((counts + _B - 1) // _B) * _B
    starts = jnp.concatenate([jnp.zeros((1,), jnp.int32),
                              jnp.cumsum(padded).astype(jnp.int32)])
    cexcl = jnp.concatenate([jnp.zeros((1,), jnp.int32),
                             jnp.cumsum(counts).astype(jnp.int32)])[:_E]
    order = jnp.argsort(flat_e, stable=True).astype(jnp.int32)
    seg = flat_e[order]
    dest_sorted = starts[seg] + (jnp.arange(_T * _K, dtype=jnp.int32) - cexcl[seg])
    dest = jnp.zeros((_T * _K,), jnp.int32).at[order].set(dest_sorted)
    src = jnp.zeros((_P,), jnp.int32).at[dest_sorted].set(order // _K)
    nvalid = starts[_E] // _B
    block_expert = jnp.minimum(
        jnp.searchsorted(starts[1:], jnp.arange(_NB, dtype=jnp.int32) * _B,
                         side='right'), _E - 1).astype(jnp.int32)
    meta = jnp.concatenate([block_expert, nvalid[None].astype(jnp.int32)])

    xs = x[src]
    ys = _grouped_ffn(meta, xs, W1, b1, W2, b2)

    wflat = w.reshape(-1)
    gathered = ys[dest]
    out = (gathered * wflat[:, None]).reshape(_T, _K, _H).sum(axis=1)
    return out


# speedup vs baseline: 2.5300x; 1.2167x over previous
import functools
import jax
import jax.numpy as jnp
from jax.experimental import pallas as pl
from jax.experimental.pallas import tpu as pltpu

_H = 1024
_E = 8
_K = 2
_F = 4096
_T = 4096

_B = 256
_P = _T * _K + _E * _B
_NB = _P // _B
_FB = 1024
_NF = _F // _FB

_NEG = -3e38


def _gating_body(x_ref, wg_ref, bg_ref, out_ref):
    s = jnp.dot(x_ref[...], wg_ref[...], preferred_element_type=jnp.float32)
    s = s + bg_ref[...]
    col = jax.lax.broadcasted_iota(jnp.int32, s.shape, 1)
    s = jnp.where(col < _E, s, _NEG)
    m1 = jnp.max(s, axis=1, keepdims=True)
    i1 = jnp.min(jnp.where(s == m1, col, 128), axis=1, keepdims=True)
    s2 = jnp.where(col == i1, _NEG, s)
    m2 = jnp.max(s2, axis=1, keepdims=True)
    i2 = jnp.min(jnp.where(s2 == m2, col, 128), axis=1, keepdims=True)
    z = jnp.exp(m2 - m1)
    w1 = 1.0 / (1.0 + z)
    w2 = z / (1.0 + z)
    out = jnp.where(col == 0, i1.astype(jnp.float32),
          jnp.where(col == 1, i2.astype(jnp.float32),
          jnp.where(col == 2, w1,
          jnp.where(col == 3, w2, 0.0))))
    out_ref[...] = out


def _gating(x, Wg, bg):
    tb = 1024
    wgp = jnp.zeros((_H, 128), jnp.float32).at[:, :_E].set(Wg)
    bgp = jnp.zeros((1, 128), jnp.float32).at[0, :_E].set(bg)
    out = pl.pallas_call(
        _gating_body,
        grid=(_T // tb,),
        in_specs=[
            pl.BlockSpec((tb, _H), lambda i: (i, 0)),
            pl.BlockSpec((_H, 128), lambda i: (0, 0)),
            pl.BlockSpec((1, 128), lambda i: (0, 0)),
        ],
        out_specs=pl.BlockSpec((tb, 128), lambda i: (i, 0)),
        out_shape=jax.ShapeDtypeStruct((_T, 128), jnp.float32),
    )(x, wgp, bgp)
    idx = out[:, :2].astype(jnp.int32)
    w = out[:, 2:4]
    return idx, w


def _ffn1_body(meta_ref, xs_ref, w1_ref, b1_ref, h_ref):
    b = pl.program_id(0)
    nvalid = meta_ref[_NB]

    @pl.when(b < nvalid)
    def _():
        h = jnp.dot(xs_ref[...], w1_ref[0], preferred_element_type=jnp.float32)
        h_ref[...] = jnp.maximum(h + b1_ref[0], 0.0)


def _ffn2_body(meta_ref, h_ref, w2_ref, b2_ref, ys_ref):
    b = pl.program_id(0)
    nvalid = meta_ref[_NB]

    @pl.when(b < nvalid)
    def _():
        acc = jnp.dot(h_ref[...], w2_ref[0], preferred_element_type=jnp.float32)
        ys_ref[...] = acc + b2_ref[0]


def _grouped_ffn(meta, xs, W1, b1, W2, b2):
    gs1 = pltpu.PrefetchScalarGridSpec(
        num_scalar_prefetch=1,
        grid=(_NB,),
        in_specs=[
            pl.BlockSpec((_B, _H), lambda b, m: (b, 0)),
            pl.BlockSpec((1, _H, _F), lambda b, m: (m[b], 0, 0)),
            pl.BlockSpec((1, 1, _F), lambda b, m: (m[b], 0, 0)),
        ],
        out_specs=pl.BlockSpec((_B, _F), lambda b, m: (b, 0)),
    )
    h = pl.pallas_call(
        _ffn1_body,
        grid_spec=gs1,
        out_shape=jax.ShapeDtypeStruct((_P, _F), jnp.float32),
    )(meta, xs, W1, b1.reshape(_E, 1, _F))

    gs2 = pltpu.PrefetchScalarGridSpec(
        num_scalar_prefetch=1,
        grid=(_NB,),
        in_specs=[
            pl.BlockSpec((_B, _F), lambda b, m: (b, 0)),
            pl.BlockSpec((1, _F, _H), lambda b, m: (m[b], 0, 0)),
            pl.BlockSpec((1, 1, _H), lambda b, m: (m[b], 0, 0)),
        ],
        out_specs=pl.BlockSpec((_B, _H), lambda b, m: (b, 0)),
    )
    return pl.pallas_call(
        _ffn2_body,
        grid_spec=gs2,
        out_shape=jax.ShapeDtypeStruct((_P, _H), jnp.float32),
    )(meta, h, W2, b2.reshape(_E, 1, _H))


@jax.jit
def kernel(x, Wg, bg, W1, b1, W2, b2):
    idx, w = _gating(x, Wg, bg)

    flat_e = idx.reshape(-1)
    counts = jnp.sum(flat_e[:, None] == jnp.arange(_E)[None, :], axis=0)
    padded = ((counts + _B - 1) // _B) * _B
    starts = jnp.concatenate([jnp.zeros((1,), jnp.int32),
                              jnp.cumsum(padded).astype(jnp.int32)])
    cexcl = jnp.concatenate([jnp.zeros((1,), jnp.int32),
                             jnp.cumsum(counts).astype(jnp.int32)])[:_E]
    order = jnp.argsort(flat_e, stable=True).astype(jnp.int32)
    seg = flat_e[order]
    dest_sorted = starts[seg] + (jnp.arange(_T * _K, dtype=jnp.int32) - cexcl[seg])
    dest = jnp.zeros((_T * _K,), jnp.int32).at[order].set(dest_sorted)
    src = jnp.zeros((_P,), jnp.int32).at[dest_sorted].set(order // _K)
    nvalid = starts[_E] // _B
    block_expert = jnp.minimum(
        jnp.searchsorted(starts[1:], jnp.arange(_NB, dtype=jnp.int32) * _B,
                         side='right'), _E - 1).astype(jnp.int32)
    meta = jnp.concatenate([block_expert, nvalid[None].astype(jnp.int32)])

    xs = x[src]
    ys = _grouped_ffn(meta, xs, W1, b1, W2, b2)

    wflat = w.reshape(-1)
    gathered = ys[dest]
    out = (gathered * wflat[:, None]).reshape(_T, _K, _H).sum(axis=1)
    return out
